# baseline (speedup 1.0000x reference)
import functools

import jax
import jax.numpy as jnp
from jax import lax
from jax.experimental import pallas as pl
from jax.experimental.pallas import tpu as pltpu
from jax.experimental.pallas import tpu_sc as plsc

_KPAD = 128


def _idx_body(p_ref, lrows_ref, hl2_ref, idx_ref):
    p = p_ref[...]
    lrows = lrows_ref[...]
    scoresT = hl2_ref[...] - jax.lax.dot_general(
        lrows, p, (((1,), (1,)), ((), ())),
        preferred_element_type=jnp.float32)
    m = jnp.min(scoresT, axis=0, keepdims=True)
    rows = jax.lax.broadcasted_iota(
        jnp.int32, scoresT.shape, 0).astype(jnp.float32)
    idx = jnp.min(jnp.where(scoresT == m, rows, float(_KPAD)), axis=0,
                  keepdims=True)
    idx_ref[...] = idx.astype(jnp.int32)


@functools.partial(jax.jit, static_argnames=("block",))
def _nearest_idx(params, lattice_points, block=16384):
    n, d = params.shape
    k = lattice_points.shape[0]
    lrows = jnp.zeros((_KPAD, d), jnp.float32).at[:k].set(lattice_points)
    hl2 = 0.5 * jnp.sum(lrows * lrows, axis=1)
    hl2 = jnp.where(jnp.arange(_KPAD) < k, hl2, jnp.inf)[:, None]
    grid = (n // block,)
    idx2d = pl.pallas_call(
        _idx_body,
        grid=grid,
        in_specs=[
            pl.BlockSpec((block, d), lambda i: (i, 0)),
            pl.BlockSpec((_KPAD, d), lambda i: (0, 0)),
            pl.BlockSpec((_KPAD, 1), lambda i: (0, 0)),
        ],
        out_specs=pl.BlockSpec((1, block), lambda i: (0, i)),
        out_shape=jax.ShapeDtypeStruct((1, n), jnp.int32),
    )(params, lrows, hl2)
    return idx2d.reshape(n)


_NC = 2
_NS = 16
_NW = _NC * _NS
_CHUNK = 512


def _make_sc_gather(n, d, k):
    per_w = n // _NW
    nch = per_w // _CHUNK
    tflat = k * d
    mesh = plsc.VectorSubcoreMesh(core_axis_name="c", subcore_axis_name="s")

    @functools.partial(
        pl.kernel, mesh=mesh,
        out_type=jax.ShapeDtypeStruct((n, d), jnp.float32),
        scratch_types=[
            pltpu.VMEM((tflat,), jnp.float32),
            pltpu.VMEM((_CHUNK,), jnp.int32),
            pltpu.VMEM((_CHUNK, d), jnp.float32),
        ],
        compiler_params=pltpu.CompilerParams(needs_layout_passes=False),
    )
    def gather(table_hbm, idx_hbm, out_hbm, table_v, idx_v, rows_v):
        wid = lax.axis_index("s") * _NC + lax.axis_index("c")
        pltpu.sync_copy(table_hbm, table_v)
        lane = lax.iota(jnp.int32, 16)

        def chunk(c, carry):
            base = wid * per_w + c * _CHUNK
            pltpu.sync_copy(idx_hbm.at[pl.ds(base, _CHUNK)], idx_v)

            def group(j, carry2):
                iv = idx_v[pl.ds(j * 16, 16)]
                word = iv * d
                row = j * 16 + lane
                for dd in range(d):
                    v = plsc.load_gather(table_v, [word + dd])
                    plsc.store_scatter(rows_v, [row, lane * 0 + dd], v)
                return carry2

            lax.fori_loop(0, _CHUNK // 16, group, 0)
            pltpu.sync_copy(rows_v, out_hbm.at[pl.ds(base, _CHUNK)])
            return carry

        lax.fori_loop(0, nch, chunk, 0)

    return gather


def kernel(params, lattice_points):
    n, d = params.shape
    k = lattice_points.shape[0]
    idx = _nearest_idx(params, lattice_points)
    table_flat = lattice_points.reshape(k * d)
    return _make_sc_gather(n, d, k)(table_flat, idx)
